# baseline (speedup 1.0000x reference)
import functools

import jax
import jax.numpy as jnp
from jax.experimental import pallas as pl
from jax.experimental.pallas import tpu as pltpu


def _leaky(v):
    return jnp.where(v >= 0, v, 0.05 * v)


def _mlp2_block(a_ref, w1_ref, b1_ref, w2_ref, b2_ref, o_ref):
    h = jnp.dot(a_ref[...], w1_ref[...], preferred_element_type=jnp.float32)
    h = _leaky(h + b1_ref[...])
    o = jnp.dot(h, w2_ref[...], preferred_element_type=jnp.float32)
    o_ref[...] = _leaky(o + b2_ref[...])


def _mlp2(a, w1, b1, w2, b2, block_rows=512):
    n, fin = a.shape
    fmid = w1.shape[1]
    fout = w2.shape[1]
    npad = ((n + block_rows - 1) // block_rows) * block_rows
    if npad != n:
        a = jnp.pad(a, ((0, npad - n), (0, 0)))
    grid = (npad // block_rows,)
    out = pl.pallas_call(
        _mlp2_block,
        grid=grid,
        in_specs=[
            pl.BlockSpec((block_rows, fin), lambda i: (i, 0)),
            pl.BlockSpec((fin, fmid), lambda i: (0, 0)),
            pl.BlockSpec((1, fmid), lambda i: (0, 0)),
            pl.BlockSpec((fmid, fout), lambda i: (0, 0)),
            pl.BlockSpec((1, fout), lambda i: (0, 0)),
        ],
        out_specs=pl.BlockSpec((block_rows, fout), lambda i: (i, 0)),
        out_shape=jax.ShapeDtypeStruct((npad, fout), jnp.float32),
    )(a, w1, b1.reshape(1, -1), w2, b2.reshape(1, -1))
    return out[:n]


def kernel(x, edge_index, edge_attribute,
           lW1_1, lb1_1, lW1_2, lb1_2, gW1_1, gb1_1, gW1_2, gb1_2,
           lW2_1, lb2_1, lW2_2, lb2_2, gW2_1, gb2_1, gW2_2, gb2_2,
           bn1_g, bn1_b, bn2_g, bn2_b):
    n_nodes = x.shape[0]
    src = edge_index[0]
    dst = edge_index[1]
    order = jnp.argsort(dst)
    ss = src[order]
    sd = dst[order]
    xs = x[ss]
    xd = x[sd]
    rel = xs - xd

    pre1 = xs @ lW1_1[:2] + rel @ lW1_1[2:4] + lb1_1
    h1e = _leaky(pre1)
    h1e = _leaky(h1e @ lW1_2 + lb1_2)
    agg1 = jax.ops.segment_max(h1e, sd, num_segments=n_nodes)
    agg1 = jnp.where(jnp.isfinite(agg1), agg1, 0.0)
    o1 = _mlp2(agg1, gW1_1, gb1_1, gW1_2, gb1_2)
    h1 = _leaky(o1)
    mu = jnp.mean(h1, axis=0)
    var = jnp.var(h1, axis=0)
    h1 = (h1 - mu) / jnp.sqrt(var + 1e-5) * bn1_g + bn1_b

    pre2 = h1[ss] @ lW2_1[:512] + rel @ lW2_1[512:514] + lb2_1
    h2e = _leaky(pre2)
    h2e = _leaky(h2e @ lW2_2 + lb2_2)
    agg2 = jax.ops.segment_max(h2e, sd, num_segments=n_nodes)
    agg2 = jnp.where(jnp.isfinite(agg2), agg2, 0.0)
    o2 = _mlp2(agg2, gW2_1, gb2_1, gW2_2, gb2_2)
    h2 = _leaky(o2)
    mu2 = jnp.mean(h2, axis=0)
    var2 = jnp.var(h2, axis=0)
    h2 = (h2 - mu2) / jnp.sqrt(var2 + 1e-5) * bn2_g + bn2_b
    return h2
